# baseline (speedup 1.0000x reference)
import functools

import jax
import jax.numpy as jnp
from jax import lax
from jax.experimental import pallas as pl
from jax.experimental.pallas import tpu as pltpu
from jax.experimental.pallas import tpu_sc as plsc


B, S, D = 4, 8192, 1024
NC, NS = 2, 16
NW = NC * NS
ROWS_PER_W = S // NW
C = 16
N_CHUNKS = ROWS_PER_W // C
CW = C * D
N_SL = CW // 16


@functools.partial(
    pl.kernel,
    out_type=jax.ShapeDtypeStruct((B * S * D,), jnp.float32),
    mesh=plsc.VectorSubcoreMesh(core_axis_name="c", subcore_axis_name="s"),
    scratch_types=[
        pltpu.VMEM((CW,), jnp.float32),
        pltpu.VMEM((CW,), jnp.float32),
        pltpu.VMEM((CW,), jnp.float32),
        pltpu.VMEM((CW,), jnp.float32),
        pltpu.VMEM((CW,), jnp.float32),
        pltpu.SemaphoreType.DMA,
        pltpu.SemaphoreType.DMA,
        pltpu.SemaphoreType.DMA,
        pltpu.SemaphoreType.DMA,
    ],
)
def _sc_add(x_hbm, pos_hbm, out_hbm, pos_v, xb0, xb1, ob0, ob1,
            sx0, sx1, so0, so1):
    wid = lax.axis_index("s") * NC + lax.axis_index("c")
    s0 = wid * ROWS_PER_W
    xbufs = (xb0, xb1)
    obufs = (ob0, ob1)
    sxs = (sx0, sx1)
    sos = (so0, so1)

    def chunk_body(ci, _):
        pos_off = (s0 + ci * C) * D
        pltpu.sync_copy(pos_hbm.at[pl.ds(pos_off, CW)], pos_v)

        def x_off(b):
            return b * (S * D) + pos_off

        in_cp = [None] * B
        out_cp = [None] * B
        for b in range(2):
            in_cp[b] = pltpu.async_copy(
                x_hbm.at[pl.ds(x_off(b), CW)], xbufs[b], sxs[b])
        for b in range(B):
            p = b % 2
            if b >= 2:
                out_cp[b - 2].wait()
            in_cp[b].wait()
            xb, ob = xbufs[p], obufs[p]

            @plsc.parallel_loop(0, N_SL, unroll=16)
            def _add(i):
                sl = pl.ds(i * 16, 16)
                ob[sl] = xb[sl] + pos_v[sl]

            out_cp[b] = pltpu.async_copy(
                ob, out_hbm.at[pl.ds(x_off(b), CW)], sos[p])
            if b + 2 < B:
                in_cp[b + 2] = pltpu.async_copy(
                    x_hbm.at[pl.ds(x_off(b + 2), CW)], xb, sxs[p])
        out_cp[B - 2].wait()
        out_cp[B - 1].wait()
        return 0

    lax.fori_loop(0, N_CHUNKS, chunk_body, 0)


def kernel(x, pos_emb):
    out = _sc_add(x.reshape(-1), pos_emb.reshape(-1))
    return out.reshape(x.shape)


# speedup vs baseline: 4.6560x; 4.6560x over previous
import jax
import jax.numpy as jnp
from jax.experimental import pallas as pl
from jax.experimental.pallas import tpu as pltpu


B, S, D = 4, 8192, 1024
S_BLK = 2048


def _add_body(x_ref, pos_ref, o_ref):
    o_ref[...] = x_ref[...] + pos_ref[...][None, :, :]


def kernel(x, pos_emb):
    b, s, d = x.shape
    n_s = s // S_BLK
    pos = pos_emb[:s]
    return pl.pallas_call(
        _add_body,
        grid=(n_s, b),
        in_specs=[
            pl.BlockSpec((1, S_BLK, d), lambda i_s, i_b: (i_b, i_s, 0)),
            pl.BlockSpec((S_BLK, d), lambda i_s, i_b: (i_s, 0)),
        ],
        out_specs=pl.BlockSpec((1, S_BLK, d), lambda i_s, i_b: (i_b, i_s, 0)),
        out_shape=jax.ShapeDtypeStruct((b, s, d), x.dtype),
        compiler_params=pltpu.CompilerParams(
            dimension_semantics=("parallel", "arbitrary"),
            vmem_limit_bytes=128 * 1024 * 1024,
        ),
    )(x, pos)
